# baseline (speedup 1.0000x reference)
import functools

import jax
import jax.numpy as jnp
from jax import lax
from jax.experimental import pallas as pl
from jax.experimental.pallas import tpu as pltpu
from jax.experimental.pallas import tpu_sc as plsc

_DIM = 64
_NC = 2
_NS = 16
_NW = _NC * _NS


def _make_gather(nb: int, np_: int):
    assert nb % _NW == 0
    b_per_w = nb // _NW

    mesh = plsc.VectorSubcoreMesh(
        core_axis_name="c", subcore_axis_name="s",
        num_cores=_NC, num_subcores=_NS,
    )

    @functools.partial(
        pl.kernel,
        mesh=mesh,
        out_type=jax.ShapeDtypeStruct((np_, _DIM, nb), jnp.float32),
        scratch_types=[
            pltpu.VMEM((np_, 128), jnp.int32),
            [pltpu.VMEM((128,), jnp.int32) for _ in range(2)],
            [pltpu.VMEM((128, 128), jnp.float32) for _ in range(2)],
            [pltpu.VMEM((_DIM, 128), jnp.float32) for _ in range(2)],
            [pltpu.SemaphoreType.DMA for _ in range(2)],
            [pltpu.SemaphoreType.DMA for _ in range(2)],
        ],
        compiler_params=pltpu.CompilerParams(needs_layout_passes=False),
    )
    def gather(idx_hbm, table_hbm, out_hbm, idx_v, idx2, g, t, gsem, wsem):
        wid = lax.axis_index("s") * _NC + lax.axis_index("c")
        b0 = wid * b_per_w

        pltpu.sync_copy(idx_hbm.at[:, pl.ds(b0, b_per_w)], idx_v)

        jvec = [lax.iota(jnp.int32, 16) + 16 * q for q in range(8)]

        def prep_and_gather(p, b):
            for q in range(8):
                v = idx_v[p, pl.ds(16 * q, 16)]
                idx2[b][pl.ds(16 * q, 16)] = lax.shift_right_logical(v, 1)
            pltpu.async_copy(table_hbm.at[idx2[b]], g[b], gsem[b])

        def wait_gather(b):
            pltpu.make_async_copy(
                table_hbm.at[idx2[b]], g[b], gsem[b]).wait()

        def transpose(p, b):
            for q in range(8):
                par = (idx_v[p, pl.ds(16 * q, 16)] & 1) * _DIM
                for c in range(_DIM):
                    vals = plsc.load_gather(g[b], [jvec[q], par + c])
                    t[b][c, pl.ds(16 * q, 16)] = vals

        def start_wb(p, b):
            pltpu.async_copy(
                t[b], out_hbm.at[p, :, pl.ds(b0, b_per_w)], wsem[b])

        def wait_wb(b):
            pltpu.make_async_copy(
                t[b], out_hbm.at[0, :, pl.ds(b0, b_per_w)], wsem[b]).wait()

        prep_and_gather(0, 0)

        def body(p, carry):
            b = lax.rem(p, 2)
            for bb in range(2):
                @pl.when(b == bb)
                def _():
                    prep_and_gather(p + 1, 1 - bb)
                    wait_gather(bb)

                    @pl.when(p >= 2)
                    def _():
                        wait_wb(bb)
                    transpose(p, bb)
                    start_wb(p, bb)
            return carry

        lax.fori_loop(0, np_ - 1, body, 0)

        last = np_ - 1
        bb = last % 2
        wait_gather(bb)
        wait_wb(bb)
        transpose(last, bb)
        start_wb(last, bb)
        wait_wb(1 - bb)
        wait_wb(bb)

    return gather


def kernel(token_ids, embedding):
    nb, np_ = token_ids.shape
    table2 = embedding.reshape(embedding.shape[0] // 2, 128)
    idx_t = token_ids.T.astype(jnp.int32)
    out = _make_gather(nb, np_)(idx_t, table2)
    return out.transpose(2, 0, 1)


# speedup vs baseline: 2.0046x; 2.0046x over previous
import functools

import jax
import jax.numpy as jnp
from jax import lax
from jax.experimental import pallas as pl
from jax.experimental.pallas import tpu as pltpu
from jax.experimental.pallas import tpu_sc as plsc

_DIM = 64
_NC = 2
_NS = 16
_NW = _NC * _NS
_NBUF = 4


def _make_gather(n_rows: int, row_len: int):
    assert n_rows % _NW == 0
    rows_per_w = n_rows // _NW
    per_w = rows_per_w * row_len
    assert rows_per_w % _NBUF == 0 and row_len % 8 == 0
    n_groups = rows_per_w // _NBUF

    mesh = plsc.VectorSubcoreMesh(
        core_axis_name="c", subcore_axis_name="s",
        num_cores=_NC, num_subcores=_NS,
    )

    @functools.partial(
        pl.kernel,
        mesh=mesh,
        out_type=jax.ShapeDtypeStruct((n_rows, row_len, 128), jnp.float32),
        scratch_types=[
            pltpu.VMEM((rows_per_w, row_len), jnp.int32),
            [pltpu.VMEM((row_len, _DIM), jnp.float32) for _ in range(_NBUF)],
            [pltpu.SemaphoreType.DMA for _ in range(_NBUF)],
            [pltpu.SemaphoreType.DMA for _ in range(_NBUF)],
        ],
        compiler_params=pltpu.CompilerParams(use_tc_tiling_on_sc=False),
    )
    def gather(idx_hbm, table_hbm, out_hbm, idx_v, rows, gsem, wsem):
        wid = lax.axis_index("s") * _NC + lax.axis_index("c")
        row0 = wid * rows_per_w

        pltpu.sync_copy(idx_hbm.at[pl.ds(wid * rows_per_w, rows_per_w), :],
                        idx_v)

        def start_gather(r, b):
            pltpu.async_copy(table_hbm.at[idx_v.at[r]], rows[b], gsem[b])

        def wait_gather(b):
            pltpu.make_async_copy(
                table_hbm.at[idx_v.at[0]], rows[b], gsem[b]).wait()

        def start_wb(r, b):
            pltpu.async_copy(rows[b], out_hbm.at[row0 + r, :, pl.ds(0, _DIM)], wsem[b])

        def wait_wb(b):
            pltpu.make_async_copy(rows[b], out_hbm.at[0, :, pl.ds(0, _DIM)], wsem[b]).wait()

        for b in range(_NBUF):
            start_gather(b, b)

        def body(j, carry):
            a = j * _NBUF
            for b in range(_NBUF):
                wait_gather(b)
                start_wb(a + b, b)
            for b in range(_NBUF):
                wait_wb(b)
                start_gather(a + _NBUF + b, b)
            return carry

        lax.fori_loop(0, n_groups - 1, body, 0)

        a = (n_groups - 1) * _NBUF
        for b in range(_NBUF):
            wait_gather(b)
            start_wb(a + b, b)
        for b in range(_NBUF):
            wait_wb(b)

    return gather


def kernel(token_ids, embedding):
    b, s = token_ids.shape
    out = _make_gather(b, s)(token_ids.astype(jnp.int32), embedding)
    return out[:, :, :_DIM]
